# baseline (speedup 1.0000x reference)
import functools

import jax
import jax.numpy as jnp
from jax.experimental import pallas as pl
from jax.experimental.pallas import tpu as pltpu
from jax.experimental.pallas import tpu_sc as plsc

_EMBED = 512
_N = 16384
_NW = 16
_CHUNK = _N // _NW


@functools.partial(
    pl.kernel,
    mesh=plsc.VectorSubcoreMesh(core_axis_name="c", subcore_axis_name="s", num_cores=1),
    out_type=jax.ShapeDtypeStruct((_NW, 4 * 128), jnp.float32),
    scratch_types=[
        pltpu.VMEM((4, _CHUNK), jnp.int32),
        pltpu.VMEM((4 * 128,), jnp.float32),
        pltpu.SemaphoreType.DMA,
        pltpu.SemaphoreType.DMA,
        pltpu.SemaphoreType.DMA,
        pltpu.SemaphoreType.DMA,
    ],
    compiler_params=pltpu.CompilerParams(needs_layout_passes=False),
)
def _sc_hist(pitch_hbm, vel_hbm, prog_hbm, drum_hbm, out_hbm, idx_v, hist_v,
             sem0, sem1, sem2, sem3):
    wid = jax.lax.axis_index("s") * 1 + jax.lax.axis_index("c")
    base = wid * _CHUNK
    sems = (sem0, sem1, sem2, sem3)
    copies = [
        pltpu.async_copy(src.at[pl.ds(base, _CHUNK)], idx_v.at[t], sems[t])
        for t, src in enumerate((pitch_hbm, vel_hbm, prog_hbm, drum_hbm))
    ]
    zeros16 = jnp.zeros((16,), jnp.float32)
    ones16 = jnp.ones((16,), jnp.float32)

    def zero_blk(blk, _):
        hist_v[pl.ds(blk * 16, 16)] = zeros16
        return 0

    jax.lax.fori_loop(0, 4 * 128 // 16, zero_blk, 0)
    for c in copies:
        c.wait()
    off1 = jnp.full((16,), 128, jnp.int32)
    off2 = jnp.full((16,), 256, jnp.int32)

    def body(k, acc):
        for u in range(4):
            sl = pl.ds(k * 64 + u * 16, 16)
            plsc.addupdate_scatter(hist_v, [idx_v[0, sl]], ones16)
            plsc.addupdate_scatter(hist_v, [idx_v[1, sl] + off1], ones16)
            plsc.addupdate_scatter(hist_v, [idx_v[2, sl] + off2], ones16)
            acc = acc + idx_v[3, sl]
        return acc

    acc = jax.lax.fori_loop(0, _CHUNK // 64, body,
                            jnp.zeros((16,), jnp.int32))
    ones_cnt = jnp.sum(acc).astype(jnp.float32)
    lane = jax.lax.iota(jnp.int32, 16)
    drum_vec = jnp.where(lane == 0, _CHUNK - ones_cnt,
                         jnp.where(lane == 1, ones_cnt, 0.0))
    hist_v[pl.ds(384, 16)] = drum_vec
    pltpu.sync_copy(hist_v, out_hbm.at[wid])


def _combine_body(cnt_ref, cont_ref, wp_ref, wv_ref, wg_ref, wd_ref, out_ref):
    f32 = jnp.float32
    cnt = jnp.sum(cnt_ref[...], axis=0, keepdims=True)

    def bag(counts, w):
        return jax.lax.dot_general(
            counts, w, (((1,), (0,)), ((), ())),
            precision=jax.lax.Precision.HIGHEST,
            preferred_element_type=f32)

    pitch_bag = bag(cnt[:, 0:128], wp_ref[...])
    vel_bag = bag(cnt[:, 128:256], wv_ref[...])
    prog_bag = bag(cnt[:, 256:384], wg_ref[...])
    drum_bag = bag(cnt[:, 384:386], wd_ref[...])

    i2 = jax.lax.broadcasted_iota(jnp.int32, (1, 256), 1).astype(f32)
    freqs = jnp.exp((2.0 * i2 / _EMBED) * jnp.log(10000.0).astype(f32))
    t0 = cont_ref[0, 0] * freqs
    t1 = cont_ref[0, 1] * freqs
    time0 = jnp.concatenate([jnp.sin(t0), jnp.cos(t0)], axis=1)
    time1 = jnp.concatenate([jnp.sin(t1), jnp.cos(t1)], axis=1)

    out_ref[:, 0:512] = drum_bag
    out_ref[:, 512:1024] = time0
    out_ref[:, 1024:1536] = time1
    out_ref[:, 1536:2048] = prog_bag
    out_ref[:, 2048:2560] = pitch_bag
    out_ref[:, 2560:3072] = vel_bag


def kernel(pitch_indices, velocity_indices, program_indices,
           continuous_features, drum_indices,
           W_pitch, W_velocity, W_program, W_drum):
    partial_counts = _sc_hist(pitch_indices, velocity_indices,
                              program_indices, drum_indices)
    cont = continuous_features.reshape(1, 2)
    out = pl.pallas_call(
        _combine_body,
        out_shape=jax.ShapeDtypeStruct((1, 6 * _EMBED), jnp.float32),
    )(partial_counts, cont, W_pitch, W_velocity, W_program, W_drum)
    return out
